# Initial kernel scaffold; baseline (speedup 1.0000x reference)
import jax
import jax.numpy as jnp
from jax.experimental import pallas as pl


def kernel(x_nodes, edge_index, edge_attr, location, batch, Wn, bn, We, be, Wq, bq, Wk, bk, Wv, bv, Wed, bed, Ws, bs, W1, b1, W2, b2):
    raise NotImplementedError("write your pallas kernel here")


# speedup vs baseline: 1.1082x; 1.1082x over previous
import functools

import jax
import jax.numpy as jnp
from jax.experimental import pallas as pl

N = 10000
E = 320000
DF = 128
DE = 16
D = 64
NL = 24
B = 16


def _pre_body(x_ref, wn_ref, bn_ref, wq_ref, bq_ref, wk_ref, bk_ref,
              wv_ref, bv_ref, ws_ref, bs_ref,
              q_ref, k_ref, v_ref, skip_ref):
    x = x_ref[...]
    h = jnp.dot(x, wn_ref[...], preferred_element_type=jnp.float32) + bn_ref[...]
    q_ref[...] = jnp.dot(h, wq_ref[...], preferred_element_type=jnp.float32) + bq_ref[...]
    k_ref[...] = jnp.dot(h, wk_ref[...], preferred_element_type=jnp.float32) + bk_ref[...]
    v_ref[...] = jnp.dot(h, wv_ref[...], preferred_element_type=jnp.float32) + bv_ref[...]
    skip_ref[...] = h + jnp.dot(h, ws_ref[...], preferred_element_type=jnp.float32) + bs_ref[...]


def _ee_body(ea_ref, wf_ref, bf_ref, ee_ref):
    ee_ref[...] = jnp.dot(ea_ref[...], wf_ref[...],
                          preferred_element_type=jnp.float32) + bf_ref[...]


def _dense_pre(x_nodes, edge_attr, Wn, bn, Wq, bq, Wk, bk, Wv, bv, Ws, bs,
               Wfold, bfold):
    nb = 1000
    grid = (N // nb,)
    full = lambda shape: pl.BlockSpec(shape, lambda i: (0,) * len(shape))
    row = lambda d: pl.BlockSpec((nb, d), lambda i: (i, 0))
    q, k, v, skip = pl.pallas_call(
        _pre_body,
        grid=grid,
        in_specs=[row(DF), full((DF, D)), full((D,)),
                  full((D, D)), full((D,)), full((D, D)), full((D,)),
                  full((D, D)), full((D,)), full((D, D)), full((D,))],
        out_specs=[row(D), row(D), row(D), row(D)],
        out_shape=[jax.ShapeDtypeStruct((N, D), jnp.float32)] * 4,
    )(x_nodes, Wn, bn, Wq, bq, Wk, bk, Wv, bv, Ws, bs)

    ebk = 4000
    ee = pl.pallas_call(
        _ee_body,
        grid=(E // ebk,),
        in_specs=[pl.BlockSpec((ebk, DE), lambda i: (i, 0)),
                  full((DE, D)), full((D,))],
        out_specs=pl.BlockSpec((ebk, D), lambda i: (i, 0)),
        out_shape=jax.ShapeDtypeStruct((E, D), jnp.float32),
    )(edge_attr, Wfold, bfold)
    return q, k, v, skip, ee


def kernel(x_nodes, edge_index, edge_attr, location, batch,
           Wn, bn, We, be, Wq, bq, Wk, bk, Wv, bv, Wed, bed, Ws, bs,
           W1, b1, W2, b2):
    Wfold = We @ Wed
    bfold = be @ Wed + bed
    q, k, v, skip, ee = _dense_pre(x_nodes, edge_attr, Wn, bn, Wq, bq,
                                   Wk, bk, Wv, bv, Ws, bs, Wfold, bfold)
    src = edge_index[0]
    dst = edge_index[1]
    k_j = k[src] + ee
    v_j = v[src] + ee
    q_i = q[dst]
    alpha = jnp.sum(q_i * k_j, axis=-1) / jnp.sqrt(float(D))
    m = jax.ops.segment_max(alpha, dst, num_segments=N)
    alpha = jnp.exp(alpha - m[dst])
    denom = jax.ops.segment_sum(alpha, dst, num_segments=N)
    alpha = alpha / (denom[dst] + 1e-16)
    out = jax.ops.segment_sum(alpha[:, None] * v_j, dst, num_segments=N)
    out = out + skip
    out = jnp.maximum(out, 0.0)
    seg = location + NL * batch
    S = NL * B
    ssum = jax.ops.segment_sum(out, seg, num_segments=S)
    cnt = jax.ops.segment_sum(jnp.ones((N,), jnp.float32), seg, num_segments=S)
    g = ssum / jnp.maximum(cnt, 1.0)[:, None]
    g = jnp.maximum(g @ W1 + b1, 0.0)
    return g @ W2 + b2


# speedup vs baseline: 5.7912x; 5.2256x over previous
import functools

import jax
import jax.numpy as jnp
from jax import lax
from jax.experimental import pallas as pl
from jax.experimental.pallas import tpu as pltpu
from jax.experimental.pallas import tpu_sc as plsc

N = 10000
E = 320000
DF = 128
DE = 16
D = 64
NL = 24
B = 16
S = NL * B

NW = 32
EW = E // NW
C = 80
NCHUNK = EW // C
ACCW = 128
ROWS = 632
NPAD = 16 * ROWS


def _pre_body(x_ref, wn_ref, bn_ref, wq_ref, bq_ref, wk_ref, bk_ref,
              wv_ref, bv_ref, ws_ref, bs_ref,
              q_ref, kv_ref, skip_ref):
    h = jnp.dot(x_ref[...], wn_ref[...],
                preferred_element_type=jnp.float32) + bn_ref[...]
    q_ref[:, :D] = jnp.dot(h, wq_ref[...], preferred_element_type=jnp.float32) + bq_ref[...]
    q_ref[:, D:] = jnp.zeros_like(q_ref[:, D:])
    kv_ref[:, :D] = jnp.dot(h, wk_ref[...], preferred_element_type=jnp.float32) + bk_ref[...]
    kv_ref[:, D:] = jnp.dot(h, wv_ref[...], preferred_element_type=jnp.float32) + bv_ref[...]
    skip_ref[...] = h + jnp.dot(h, ws_ref[...], preferred_element_type=jnp.float32) + bs_ref[...]


def _ee_body(ea_ref, wf_ref, bf_ref, ee_ref):
    ee_ref[...] = jnp.dot(ea_ref[...], wf_ref[...],
                          preferred_element_type=jnp.float32) + bf_ref[...]


def _dense_pre(x_nodes, edge_attr, Wn, bn, Wq, bq, Wk, bk, Wv, bv, Ws, bs,
               Wfold, bfold):
    nb = 1000
    full = lambda shape: pl.BlockSpec(shape, lambda i: (0,) * len(shape))
    q, kv, skip = pl.pallas_call(
        _pre_body,
        grid=(N // nb,),
        in_specs=[pl.BlockSpec((nb, DF), lambda i: (i, 0)),
                  full((DF, D)), full((D,)),
                  full((D, D)), full((D,)), full((D, D)), full((D,)),
                  full((D, D)), full((D,)), full((D, D)), full((D,))],
        out_specs=[pl.BlockSpec((nb, 2 * D), lambda i: (i, 0)),
                   pl.BlockSpec((nb, 2 * D), lambda i: (i, 0)),
                   pl.BlockSpec((nb, D), lambda i: (i, 0))],
        out_shape=[jax.ShapeDtypeStruct((N, 2 * D), jnp.float32),
                   jax.ShapeDtypeStruct((N, 2 * D), jnp.float32),
                   jax.ShapeDtypeStruct((N, D), jnp.float32)],
    )(x_nodes, Wn, bn, Wq, bq, Wk, bk, Wv, bv, Ws, bs)

    ebk = 4000
    ea2 = edge_attr.reshape(E // 2, 2 * DE)
    Wfold2 = jnp.zeros((2 * DE, 2 * D), Wfold.dtype)
    Wfold2 = Wfold2.at[:DE, :D].set(Wfold).at[DE:, D:].set(Wfold)
    bfold2 = jnp.concatenate([bfold, bfold])
    ee2 = pl.pallas_call(
        _ee_body,
        grid=(E // 2 // ebk,),
        in_specs=[pl.BlockSpec((ebk, 2 * DE), lambda i: (i, 0)),
                  full((2 * DE, 2 * D)), full((2 * D,))],
        out_specs=pl.BlockSpec((ebk, 2 * D), lambda i: (i, 0)),
        out_shape=jax.ShapeDtypeStruct((E // 2, 2 * D), jnp.float32),
    )(ea2, Wfold2, bfold2)
    return q, kv, skip, ee2


def _edge_sc_body(src_hbm, dst_hbm, q_hbm, kv_hbm, ee_hbm, zer_hbm,
                  out_hbm,
                  sidx, didx, qb, kvb, eb, mb, acc_sh, sem):
    c = lax.axis_index("c")
    s = lax.axis_index("s")
    wid = s * 2 + c

    base = s * ROWS
    pltpu.sync_copy(zer_hbm.at[pl.ds(base, ROWS)], acc_sh.at[pl.ds(base, ROWS)])

    zeros16 = jnp.zeros((16,), jnp.float32)

    def zrow(ei, carry):
        for t in range(5, 8):
            mb[ei, pl.ds(16 * t, 16)] = zeros16
        return carry

    lax.fori_loop(0, C, zrow, 0)
    plsc.subcore_barrier()

    lane0 = jnp.where(lax.iota(jnp.int32, 16) == 0, 1.0, 0.0)
    e0 = wid * EW
    _GDN = lax.GatherDimensionNumbers(offset_dims=(), collapsed_slice_dims=(0,),
                                      start_index_map=(0,))
    lanes = lax.iota(jnp.int32, 16)
    perms = [(lanes ^ sh)[:, None] for sh in (8, 4, 2, 1)]

    def chunk(i, carry):
        off = pl.multiple_of(e0 + i * C, 16)
        pltpu.sync_copy(src_hbm.at[pl.ds(off, C)], sidx)
        pltpu.sync_copy(dst_hbm.at[pl.ds(off, C)], didx)
        pltpu.async_copy(q_hbm.at[didx], qb, sem).wait()
        pltpu.async_copy(kv_hbm.at[sidx], kvb, sem).wait()
        eoff = pl.multiple_of(off // 2, 8)
        pltpu.sync_copy(ee_hbm.at[pl.ds(eoff, C * D // 128)], eb)

        def edge(ei, carry2):
            erow = ei // 2
            ecol = (ei % 2) * D
            acc = jnp.zeros((16,), jnp.float32)
            evs = []
            for t in range(4):
                sl = pl.ds(16 * t, 16)
                ev = eb[erow, pl.ds(ecol + 16 * t, 16)]
                evs.append(ev)
                acc = acc + qb[ei, sl] * (kvb[ei, sl] + ev)
            for p in perms:
                acc = acc + lax.gather(
                    acc, p, _GDN, (1,),
                    mode=lax.GatherScatterMode.PROMISE_IN_BOUNDS)
            w = jnp.exp(acc * 0.125)
            for t in range(4):
                mb[ei, pl.ds(16 * t, 16)] = w * (kvb[ei, pl.ds(D + 16 * t, 16)] + evs[t])
            mb[ei, pl.ds(64, 16)] = w * lane0
            return carry2

        lax.fori_loop(0, C, edge, 0)
        pltpu.sync_copy(mb, acc_sh.at[didx], add=True)
        return carry

    lax.fori_loop(0, NCHUNK, chunk, 0)
    plsc.subcore_barrier()
    pltpu.sync_copy(acc_sh.at[pl.ds(base, ROWS)],
                    out_hbm.at[c, pl.ds(base, ROWS)])


def _edge_sc(src, dst, q, kv, ee2, zer):
    mesh = plsc.VectorSubcoreMesh(core_axis_name="c", subcore_axis_name="s")
    f = functools.partial(
        pl.kernel, _edge_sc_body, mesh=mesh,
        out_type=jax.ShapeDtypeStruct((2, NPAD, ACCW), jnp.float32),
        scratch_types=[
            pltpu.VMEM((C,), jnp.int32),
            pltpu.VMEM((C,), jnp.int32),
            pltpu.VMEM((C, 2 * D), jnp.float32),
            pltpu.VMEM((C, 2 * D), jnp.float32),
            pltpu.VMEM((C * D // 128, 128), jnp.float32),
            pltpu.VMEM((C, ACCW), jnp.float32),
            pltpu.VMEM_SHARED((NPAD, ACCW), jnp.float32),
            pltpu.SemaphoreType.DMA,
        ],
    )()
    return f(src, dst, q, kv, ee2, zer)


def _post_body(a0_ref, a1_ref, skip_ref, seg_ref, w1_ref, b1_ref,
               w2_ref, b2_ref, y_ref, pool_ref, cnt_ref):
    i = pl.program_id(0)
    nb = skip_ref.shape[0]
    num = a0_ref[:, :D] + a1_ref[:, :D]
    den = a0_ref[:, D:D + 1] + a1_ref[:, D:D + 1]
    out = num / (den + 1e-16) + skip_ref[...]
    out = jnp.maximum(out, 0.0)
    seg = seg_ref[...]
    sids = lax.broadcasted_iota(jnp.int32, (nb, S), 1)
    onehot = (sids == seg).astype(jnp.float32)

    @pl.when(i == 0)
    def _():
        pool_ref[...] = jnp.zeros_like(pool_ref)
        cnt_ref[...] = jnp.zeros_like(cnt_ref)

    pool_ref[...] += lax.dot_general(onehot, out, (((0,), (0,)), ((), ())),
                                     preferred_element_type=jnp.float32)
    cnt_ref[...] += lax.dot_general(onehot, jnp.ones((nb, 1), jnp.float32),
                                    (((0,), (0,)), ((), ())),
                                    preferred_element_type=jnp.float32)

    @pl.when(i == pl.num_programs(0) - 1)
    def _():
        g = pool_ref[...] / jnp.maximum(cnt_ref[...], 1.0)
        g = jnp.maximum(jnp.dot(g, w1_ref[...],
                                preferred_element_type=jnp.float32) + b1_ref[...], 0.0)
        y_ref[...] = jnp.dot(g, w2_ref[...],
                             preferred_element_type=jnp.float32) + b2_ref[...]


def _post(a0, a1, skip, seg, W1, b1, W2, b2):
    nb = 1000
    full = lambda shape: pl.BlockSpec(shape, lambda i: (0,) * len(shape))
    return pl.pallas_call(
        _post_body,
        grid=(N // nb,),
        in_specs=[pl.BlockSpec((nb, ACCW), lambda i: (i, 0)),
                  pl.BlockSpec((nb, ACCW), lambda i: (i, 0)),
                  pl.BlockSpec((nb, D), lambda i: (i, 0)),
                  pl.BlockSpec((nb, 1), lambda i: (i, 0)),
                  full((D, 2 * D)), full((2 * D,)),
                  full((2 * D, 1)), full((1,))],
        out_specs=full((S, 1)),
        out_shape=jax.ShapeDtypeStruct((S, 1), jnp.float32),
        scratch_shapes=[pltpu.VMEM((S, D), jnp.float32),
                        pltpu.VMEM((S, 1), jnp.float32)],
    )(a0, a1, skip, seg, W1, b1, W2, b2)


def kernel(x_nodes, edge_index, edge_attr, location, batch,
           Wn, bn, We, be, Wq, bq, Wk, bk, Wv, bv, Wed, bed, Ws, bs,
           W1, b1, W2, b2):
    Wfold = We @ Wed
    bfold = be @ Wed + bed
    q, kv, skip, ee2 = _dense_pre(x_nodes, edge_attr, Wn, bn, Wq, bq,
                                  Wk, bk, Wv, bv, Ws, bs, Wfold, bfold)
    src = edge_index[0]
    dst = edge_index[1]
    zer = jnp.zeros((NPAD, ACCW), jnp.float32)
    acc = _edge_sc(src, dst, q, kv, ee2, zer)
    seg = (location + NL * batch).astype(jnp.int32).reshape(N, 1)
    return _post(acc[0], acc[1], skip, seg, W1, b1, W2, b2)


# speedup vs baseline: 6.6789x; 1.1533x over previous
import functools

import jax
import jax.numpy as jnp
from jax import lax
from jax.experimental import pallas as pl
from jax.experimental.pallas import tpu as pltpu
from jax.experimental.pallas import tpu_sc as plsc

N = 10000
E = 320000
DF = 128
DE = 16
D = 64
NL = 24
B = 16
S = NL * B

NW = 32
EW = E // NW
C = 80
NCHUNK = EW // C
ACCW = 128
ROWS = 632
NPAD = 16 * ROWS


def _pre_body(x_ref, wn_ref, bn_ref, wq_ref, bq_ref, wk_ref, bk_ref,
              wv_ref, bv_ref, ws_ref, bs_ref,
              q_ref, kv_ref, skip_ref):
    h = jnp.dot(x_ref[...], wn_ref[...],
                preferred_element_type=jnp.float32) + bn_ref[...]
    q_ref[:, :D] = jnp.dot(h, wq_ref[...], preferred_element_type=jnp.float32) + bq_ref[...]
    q_ref[:, D:] = jnp.zeros_like(q_ref[:, D:])
    kv_ref[:, :D] = jnp.dot(h, wk_ref[...], preferred_element_type=jnp.float32) + bk_ref[...]
    kv_ref[:, D:] = jnp.dot(h, wv_ref[...], preferred_element_type=jnp.float32) + bv_ref[...]
    skip_ref[...] = h + jnp.dot(h, ws_ref[...], preferred_element_type=jnp.float32) + bs_ref[...]


def _ee_body(ea_ref, wf_ref, bf_ref, ee_ref):
    ee_ref[...] = jnp.dot(ea_ref[...], wf_ref[...],
                          preferred_element_type=jnp.float32) + bf_ref[...]


def _dense_pre(x_nodes, edge_attr, Wn, bn, Wq, bq, Wk, bk, Wv, bv, Ws, bs,
               Wfold, bfold):
    nb = 1000
    full = lambda shape: pl.BlockSpec(shape, lambda i: (0,) * len(shape))
    q, kv, skip = pl.pallas_call(
        _pre_body,
        grid=(N // nb,),
        in_specs=[pl.BlockSpec((nb, DF), lambda i: (i, 0)),
                  full((DF, D)), full((D,)),
                  full((D, D)), full((D,)), full((D, D)), full((D,)),
                  full((D, D)), full((D,)), full((D, D)), full((D,))],
        out_specs=[pl.BlockSpec((nb, 2 * D), lambda i: (i, 0)),
                   pl.BlockSpec((nb, 2 * D), lambda i: (i, 0)),
                   pl.BlockSpec((nb, D), lambda i: (i, 0))],
        out_shape=[jax.ShapeDtypeStruct((N, 2 * D), jnp.float32),
                   jax.ShapeDtypeStruct((N, 2 * D), jnp.float32),
                   jax.ShapeDtypeStruct((N, D), jnp.float32)],
    )(x_nodes, Wn, bn, Wq, bq, Wk, bk, Wv, bv, Ws, bs)

    ebk = 4000
    ea2 = edge_attr.reshape(E // 2, 2 * DE)
    Wfold2 = jnp.zeros((2 * DE, 2 * D), Wfold.dtype)
    Wfold2 = Wfold2.at[:DE, :D].set(Wfold).at[DE:, D:].set(Wfold)
    bfold2 = jnp.concatenate([bfold, bfold])
    ee2 = pl.pallas_call(
        _ee_body,
        grid=(E // 2 // ebk,),
        in_specs=[pl.BlockSpec((ebk, 2 * DE), lambda i: (i, 0)),
                  full((2 * DE, 2 * D)), full((2 * D,))],
        out_specs=pl.BlockSpec((ebk, 2 * D), lambda i: (i, 0)),
        out_shape=jax.ShapeDtypeStruct((E // 2, 2 * D), jnp.float32),
    )(ea2, Wfold2, bfold2)
    return q, kv, skip, ee2


CF = 80
NCH = EW // CF


def _edge_sc_body(src_hbm, dst_hbm, q_hbm, kv_hbm, ee_hbm, out_hbm,
                  sidx, didx, qb, kvb, eb, mb, acc_sh, semg):
    c = lax.axis_index("c")
    s = lax.axis_index("s")
    wid = s * 2 + c
    base = s * ROWS
    e0 = wid * EW

    z16 = jnp.zeros((16,), jnp.float32)

    def zrow(j, carry):
        for t in range(ACCW // 16):
            mb[j, pl.ds(16 * t, 16)] = z16
        return carry

    lax.fori_loop(0, CF, zrow, 0)
    for ofs, ln in ((0, 80), (80, 80), (160, 80), (240, 80), (320, 80),
                    (400, 80), (480, 80), (560, 72)):
        pltpu.sync_copy(mb.at[pl.ds(0, ln)], acc_sh.at[pl.ds(base + ofs, ln)])
    plsc.subcore_barrier()

    lane0 = jnp.where(lax.iota(jnp.int32, 16) == 0, 1.0, 0.0)
    _GDN = lax.GatherDimensionNumbers(offset_dims=(), collapsed_slice_dims=(0,),
                                      start_index_map=(0,))
    lanes = lax.iota(jnp.int32, 16)
    perms = [(lanes ^ sh)[:, None] for sh in (8, 4, 2, 1)]

    def do_edge(ei, er, ec):
        acc = jnp.zeros((16,), jnp.float32)
        evs = []
        for t in range(4):
            sl = pl.ds(16 * t, 16)
            ev = eb[er, pl.ds(ec + 16 * t, 16)]
            evs.append(ev)
            acc = acc + qb[ei, sl] * (kvb[ei, sl] + ev)
        for p in perms:
            acc = acc + lax.gather(acc, p, _GDN, (1,),
                                   mode=lax.GatherScatterMode.PROMISE_IN_BOUNDS)
        w = jnp.exp(acc * 0.125)
        for t in range(4):
            mb[ei, pl.ds(16 * t, 16)] = w * (kvb[ei, pl.ds(D + 16 * t, 16)] + evs[t])
        mb[ei, pl.ds(64, 16)] = w * lane0

    def chunk(i, carry):
        off = pl.multiple_of(e0 + i * CF, 16)
        pltpu.sync_copy(src_hbm.at[pl.ds(off, CF)], sidx)
        pltpu.sync_copy(dst_hbm.at[pl.ds(off, CF)], didx)
        pltpu.async_copy(q_hbm.at[didx], qb, semg).wait()
        pltpu.async_copy(kv_hbm.at[sidx], kvb, semg).wait()
        eoff = pl.multiple_of(off // 2, 8)
        pltpu.sync_copy(ee_hbm.at[pl.ds(eoff, CF // 2)], eb)

        def quad(j, carry2):
            b4 = j * 4
            r2 = j * 2
            for u in range(4):
                do_edge(b4 + u, r2 + u // 2, (u % 2) * D)
            return carry2

        lax.fori_loop(0, CF // 4, quad, 0)
        pltpu.sync_copy(mb, acc_sh.at[didx], add=True)
        return carry

    lax.fori_loop(0, NCH, chunk, 0)
    plsc.subcore_barrier()
    pltpu.sync_copy(acc_sh.at[pl.ds(base, ROWS)],
                    out_hbm.at[c, pl.ds(base, ROWS)])


def _edge_sc(src, dst, q, kv, ee2):
    mesh = plsc.VectorSubcoreMesh(core_axis_name="c", subcore_axis_name="s")
    f = functools.partial(
        pl.kernel, _edge_sc_body, mesh=mesh,
        out_type=jax.ShapeDtypeStruct((2, NPAD, ACCW), jnp.float32),
        scratch_types=[
            pltpu.VMEM((CF,), jnp.int32),
            pltpu.VMEM((CF,), jnp.int32),
            pltpu.VMEM((CF, 2 * D), jnp.float32),
            pltpu.VMEM((CF, 2 * D), jnp.float32),
            pltpu.VMEM((CF // 2, 2 * D), jnp.float32),
            pltpu.VMEM((CF, ACCW), jnp.float32),
            pltpu.VMEM_SHARED((NPAD, ACCW), jnp.float32),
            pltpu.SemaphoreType.DMA,
        ],
    )()
    return f(src, dst, q, kv, ee2)


def _post_body(a0_ref, a1_ref, skip_ref, seg_ref, w1_ref, b1_ref,
               w2_ref, b2_ref, y_ref, pool_ref, cnt_ref):
    i = pl.program_id(0)
    nb = skip_ref.shape[0]
    num = a0_ref[:, :D] + a1_ref[:, :D]
    den = a0_ref[:, D:D + 1] + a1_ref[:, D:D + 1]
    out = num / (den + 1e-16) + skip_ref[...]
    out = jnp.maximum(out, 0.0)
    seg = seg_ref[...]
    sids = lax.broadcasted_iota(jnp.int32, (nb, S), 1)
    onehot = (sids == seg).astype(jnp.float32)

    @pl.when(i == 0)
    def _():
        pool_ref[...] = jnp.zeros_like(pool_ref)
        cnt_ref[...] = jnp.zeros_like(cnt_ref)

    pool_ref[...] += lax.dot_general(onehot, out, (((0,), (0,)), ((), ())),
                                     preferred_element_type=jnp.float32)
    cnt_ref[...] += lax.dot_general(onehot, jnp.ones((nb, 1), jnp.float32),
                                    (((0,), (0,)), ((), ())),
                                    preferred_element_type=jnp.float32)

    @pl.when(i == pl.num_programs(0) - 1)
    def _():
        g = pool_ref[...] / jnp.maximum(cnt_ref[...], 1.0)
        g = jnp.maximum(jnp.dot(g, w1_ref[...],
                                preferred_element_type=jnp.float32) + b1_ref[...], 0.0)
        y_ref[...] = jnp.dot(g, w2_ref[...],
                             preferred_element_type=jnp.float32) + b2_ref[...]


def _post(a0, a1, skip, seg, W1, b1, W2, b2):
    nb = 1000
    full = lambda shape: pl.BlockSpec(shape, lambda i: (0,) * len(shape))
    return pl.pallas_call(
        _post_body,
        grid=(N // nb,),
        in_specs=[pl.BlockSpec((nb, ACCW), lambda i: (i, 0)),
                  pl.BlockSpec((nb, ACCW), lambda i: (i, 0)),
                  pl.BlockSpec((nb, D), lambda i: (i, 0)),
                  pl.BlockSpec((nb, 1), lambda i: (i, 0)),
                  full((D, 2 * D)), full((2 * D,)),
                  full((2 * D, 1)), full((1,))],
        out_specs=full((S, 1)),
        out_shape=jax.ShapeDtypeStruct((S, 1), jnp.float32),
        scratch_shapes=[pltpu.VMEM((S, D), jnp.float32),
                        pltpu.VMEM((S, 1), jnp.float32)],
    )(a0, a1, skip, seg, W1, b1, W2, b2)


def kernel(x_nodes, edge_index, edge_attr, location, batch,
           Wn, bn, We, be, Wq, bq, Wk, bk, Wv, bv, Wed, bed, Ws, bs,
           W1, b1, W2, b2):
    Wfold = We @ Wed
    bfold = be @ Wed + bed
    q, kv, skip, ee2 = _dense_pre(x_nodes, edge_attr, Wn, bn, Wq, bq,
                                  Wk, bk, Wv, bv, Ws, bs, Wfold, bfold)
    acc = _edge_sc(edge_index[0], edge_index[1], q, kv, ee2)
    seg = (location + NL * batch).astype(jnp.int32).reshape(N, 1)
    return _post(acc[0], acc[1], skip, seg, W1, b1, W2, b2)


# speedup vs baseline: 8.4131x; 1.2597x over previous
import functools

import jax
import jax.numpy as jnp
from jax import lax
from jax.experimental import pallas as pl
from jax.experimental.pallas import tpu as pltpu
from jax.experimental.pallas import tpu_sc as plsc

N = 10000
E = 320000
DF = 128
DE = 16
D = 64
NL = 24
B = 16
S = NL * B

NW = 32
EW = E // NW
C = 80
NCHUNK = EW // C
ACCW = 128
ROWS = 632
NPAD = 16 * ROWS


def _pre_body(x_ref, wn_ref, bn_ref, wq_ref, bq_ref, wk_ref, bk_ref,
              wv_ref, bv_ref, ws_ref, bs_ref,
              q_ref, kv_ref, skip_ref):
    h = jnp.dot(x_ref[...], wn_ref[...],
                preferred_element_type=jnp.float32) + bn_ref[...]
    q_ref[:, :D] = jnp.dot(h, wq_ref[...], preferred_element_type=jnp.float32) + bq_ref[...]
    q_ref[:, D:] = jnp.zeros_like(q_ref[:, D:])
    kv_ref[:, :D] = jnp.dot(h, wk_ref[...], preferred_element_type=jnp.float32) + bk_ref[...]
    kv_ref[:, D:] = jnp.dot(h, wv_ref[...], preferred_element_type=jnp.float32) + bv_ref[...]
    skip_ref[...] = h + jnp.dot(h, ws_ref[...], preferred_element_type=jnp.float32) + bs_ref[...]


def _ee_body(ea_ref, wf_ref, bf_ref, ee_ref):
    ee_ref[...] = jnp.dot(ea_ref[...], wf_ref[...],
                          preferred_element_type=jnp.float32) + bf_ref[...]


def _dense_pre(x_nodes, edge_attr, Wn, bn, Wq, bq, Wk, bk, Wv, bv, Ws, bs,
               Wfold, bfold):
    nb = 1000
    full = lambda shape: pl.BlockSpec(shape, lambda i: (0,) * len(shape))
    q, kv, skip = pl.pallas_call(
        _pre_body,
        grid=(N // nb,),
        in_specs=[pl.BlockSpec((nb, DF), lambda i: (i, 0)),
                  full((DF, D)), full((D,)),
                  full((D, D)), full((D,)), full((D, D)), full((D,)),
                  full((D, D)), full((D,)), full((D, D)), full((D,))],
        out_specs=[pl.BlockSpec((nb, 2 * D), lambda i: (i, 0)),
                   pl.BlockSpec((nb, 2 * D), lambda i: (i, 0)),
                   pl.BlockSpec((nb, D), lambda i: (i, 0))],
        out_shape=[jax.ShapeDtypeStruct((N, 2 * D), jnp.float32),
                   jax.ShapeDtypeStruct((N, 2 * D), jnp.float32),
                   jax.ShapeDtypeStruct((N, D), jnp.float32)],
    )(x_nodes, Wn, bn, Wq, bq, Wk, bk, Wv, bv, Ws, bs)

    ebk = 4000
    ea2 = edge_attr.reshape(E // 2, 2 * DE)
    Wfold2 = jnp.zeros((2 * DE, 2 * D), Wfold.dtype)
    Wfold2 = Wfold2.at[:DE, :D].set(Wfold).at[DE:, D:].set(Wfold)
    bfold2 = jnp.concatenate([bfold, bfold])
    ee2 = pl.pallas_call(
        _ee_body,
        grid=(E // 2 // ebk,),
        in_specs=[pl.BlockSpec((ebk, 2 * DE), lambda i: (i, 0)),
                  full((2 * DE, 2 * D)), full((2 * D,))],
        out_specs=pl.BlockSpec((ebk, 2 * D), lambda i: (i, 0)),
        out_shape=jax.ShapeDtypeStruct((E // 2, 2 * D), jnp.float32),
    )(ea2, Wfold2, bfold2)
    return q, kv, skip, ee2


CF = 80
NCH = EW // CF


def _edge_sc_body(src_hbm, dst_hbm, q_hbm, kv_hbm, ee_hbm, out_hbm,
                  sidx, didx, qb, kvb, eb, mb, acc_sh, semg, semi):
    c = lax.axis_index("c")
    s = lax.axis_index("s")
    wid = s * 2 + c
    base = s * ROWS
    e0 = wid * EW

    z16 = jnp.zeros((16,), jnp.float32)

    def zrow(j, carry):
        for t in range(ACCW // 16):
            mb[j, pl.ds(16 * t, 16)] = z16
        return carry

    lax.fori_loop(0, CF, zrow, 0)
    for ofs, ln in ((0, 80), (80, 80), (160, 80), (240, 80), (320, 80),
                    (400, 80), (480, 80), (560, 72)):
        pltpu.sync_copy(mb.at[pl.ds(0, ln)], acc_sh.at[pl.ds(base + ofs, ln)])
    plsc.subcore_barrier()

    lane0 = jnp.where(lax.iota(jnp.int32, 16) == 0, 1.0, 0.0)
    _GDN = lax.GatherDimensionNumbers(offset_dims=(), collapsed_slice_dims=(0,),
                                      start_index_map=(0,))
    lanes = lax.iota(jnp.int32, 16)
    perms = [(lanes ^ sh)[:, None] for sh in (8, 4, 2, 1)]

    def do_edge(ei, er, ec):
        acc = jnp.zeros((16,), jnp.float32)
        evs = []
        for t in range(4):
            sl = pl.ds(16 * t, 16)
            ev = eb[er, pl.ds(ec + 16 * t, 16)]
            evs.append(ev)
            acc = acc + qb[ei, sl] * (kvb[ei, sl] + ev)
        for p in perms:
            acc = acc + lax.gather(acc, p, _GDN, (1,),
                                   mode=lax.GatherScatterMode.PROMISE_IN_BOUNDS)
        w = jnp.exp(acc * 0.125)
        for t in range(4):
            mb[ei, pl.ds(16 * t, 16)] = w * (kvb[ei, pl.ds(D + 16 * t, 16)] + evs[t])
        mb[ei, pl.ds(64, 16)] = w * lane0

    def chunk(i, carry):
        off = pl.multiple_of(e0 + i * CF, 16)
        h1 = pltpu.async_copy(src_hbm.at[pl.ds(off, CF)], sidx, semi)
        h2 = pltpu.async_copy(dst_hbm.at[pl.ds(off, CF)], didx, semi)
        eoff = pl.multiple_of(off // 2, 8)
        h3 = pltpu.async_copy(ee_hbm.at[pl.ds(eoff, CF // 2)], eb, semi)
        h1.wait()
        h2.wait()
        g1 = pltpu.async_copy(q_hbm.at[didx], qb, semg)
        g2 = pltpu.async_copy(kv_hbm.at[sidx], kvb, semg)
        h3.wait()
        g1.wait()
        g2.wait()

        def quad(j, carry2):
            b4 = j * 4
            r2 = j * 2
            for u in range(4):
                do_edge(b4 + u, r2 + u // 2, (u % 2) * D)
            return carry2

        lax.fori_loop(0, CF // 4, quad, 0)
        pltpu.sync_copy(mb, acc_sh.at[didx], add=True)
        return carry

    lax.fori_loop(0, NCH, chunk, 0)
    plsc.subcore_barrier()
    pltpu.sync_copy(acc_sh.at[pl.ds(base, ROWS)],
                    out_hbm.at[c, pl.ds(base, ROWS)])


def _edge_sc(src, dst, q, kv, ee2):
    mesh = plsc.VectorSubcoreMesh(core_axis_name="c", subcore_axis_name="s")
    f = functools.partial(
        pl.kernel, _edge_sc_body, mesh=mesh,
        out_type=jax.ShapeDtypeStruct((2, NPAD, ACCW), jnp.float32),
        scratch_types=[
            pltpu.VMEM((CF,), jnp.int32),
            pltpu.VMEM((CF,), jnp.int32),
            pltpu.VMEM((CF, 2 * D), jnp.float32),
            pltpu.VMEM((CF, 2 * D), jnp.float32),
            pltpu.VMEM((CF // 2, 2 * D), jnp.float32),
            pltpu.VMEM((CF, ACCW), jnp.float32),
            pltpu.VMEM_SHARED((NPAD, ACCW), jnp.float32),
            pltpu.SemaphoreType.DMA,
            pltpu.SemaphoreType.DMA,
        ],
    )()
    return f(src, dst, q, kv, ee2)


def _post_body(a0_ref, a1_ref, skip_ref, seg_ref, w1_ref, b1_ref,
               w2_ref, b2_ref, y_ref, pool_ref, cnt_ref):
    i = pl.program_id(0)
    nb = skip_ref.shape[0]
    num = a0_ref[:, :D] + a1_ref[:, :D]
    den = a0_ref[:, D:D + 1] + a1_ref[:, D:D + 1]
    out = num / (den + 1e-16) + skip_ref[...]
    out = jnp.maximum(out, 0.0)
    seg = seg_ref[...]
    sids = lax.broadcasted_iota(jnp.int32, (nb, S), 1)
    onehot = (sids == seg).astype(jnp.float32)

    @pl.when(i == 0)
    def _():
        pool_ref[...] = jnp.zeros_like(pool_ref)
        cnt_ref[...] = jnp.zeros_like(cnt_ref)

    pool_ref[...] += lax.dot_general(onehot, out, (((0,), (0,)), ((), ())),
                                     preferred_element_type=jnp.float32)
    cnt_ref[...] += lax.dot_general(onehot, jnp.ones((nb, 1), jnp.float32),
                                    (((0,), (0,)), ((), ())),
                                    preferred_element_type=jnp.float32)

    @pl.when(i == pl.num_programs(0) - 1)
    def _():
        g = pool_ref[...] / jnp.maximum(cnt_ref[...], 1.0)
        g = jnp.maximum(jnp.dot(g, w1_ref[...],
                                preferred_element_type=jnp.float32) + b1_ref[...], 0.0)
        y_ref[...] = jnp.dot(g, w2_ref[...],
                             preferred_element_type=jnp.float32) + b2_ref[...]


def _post(a0, a1, skip, seg, W1, b1, W2, b2):
    nb = 1000
    full = lambda shape: pl.BlockSpec(shape, lambda i: (0,) * len(shape))
    return pl.pallas_call(
        _post_body,
        grid=(N // nb,),
        in_specs=[pl.BlockSpec((nb, ACCW), lambda i: (i, 0)),
                  pl.BlockSpec((nb, ACCW), lambda i: (i, 0)),
                  pl.BlockSpec((nb, D), lambda i: (i, 0)),
                  pl.BlockSpec((nb, 1), lambda i: (i, 0)),
                  full((D, 2 * D)), full((2 * D,)),
                  full((2 * D, 1)), full((1,))],
        out_specs=full((S, 1)),
        out_shape=jax.ShapeDtypeStruct((S, 1), jnp.float32),
        scratch_shapes=[pltpu.VMEM((S, D), jnp.float32),
                        pltpu.VMEM((S, 1), jnp.float32)],
    )(a0, a1, skip, seg, W1, b1, W2, b2)


def kernel(x_nodes, edge_index, edge_attr, location, batch,
           Wn, bn, We, be, Wq, bq, Wk, bk, Wv, bv, Wed, bed, Ws, bs,
           W1, b1, W2, b2):
    Wfold = We @ Wed
    bfold = be @ Wed + bed
    q, kv, skip, ee2 = _dense_pre(x_nodes, edge_attr, Wn, bn, Wq, bq,
                                  Wk, bk, Wv, bv, Ws, bs, Wfold, bfold)
    acc = _edge_sc(edge_index[0], edge_index[1], q, kv, ee2)
    seg = (location + NL * batch).astype(jnp.int32).reshape(N, 1)
    return _post(acc[0], acc[1], skip, seg, W1, b1, W2, b2)
